# baseline (speedup 1.0000x reference)
import functools

import jax
import jax.numpy as jnp
from jax import lax
from jax.experimental import pallas as pl
from jax.experimental.pallas import tpu as pltpu
from jax.experimental.pallas import tpu_sc as plsc

_IGNORE = 0
_EPS = 1e-6
_BN = 131072
_NW = 32
_CPAD = 32


def _sc_hist_body(t_hbm, out_hbm, t_v, bins_v):
    wid = lax.axis_index("s") * 2 + lax.axis_index("c")
    n_per_w = t_v.shape[0]
    base = wid * n_per_w
    pltpu.sync_copy(t_hbm.at[pl.ds(base, n_per_w)], t_v)
    zeros = jnp.zeros((16,), jnp.float32)
    for r in range(_CPAD):
        bins_v[pl.ds(r * 16, 16)] = zeros
    ones = jnp.ones((16,), jnp.float32)
    lane = lax.iota(jnp.int32, 16)

    def body(j, carry):
        tv = t_v[pl.ds(j * 16, 16)]
        plsc.addupdate_scatter(bins_v, [tv * 16 + lane], ones)
        return carry

    lax.fori_loop(0, n_per_w // 16, body, 0)
    pltpu.sync_copy(bins_v, out_hbm.at[wid])


def _dice_body(x_ref, t_ref, cnt_ref, out_ref, acc_ref, *, nblocks, c):
    i = pl.program_id(0)
    cls = jax.lax.broadcasted_iota(jnp.int32, (c, 128), 0)

    acc_i = jnp.zeros((c, 128), dtype=jnp.float32)
    acc_s = jnp.zeros((c, 128), dtype=jnp.float32)
    for j in range(_BN // 128):
        e = jnp.exp(x_ref[:, j * 128:(j + 1) * 128])
        m = t_ref[j:j + 1, :] == cls
        acc_i = acc_i + jnp.where(m, e, 0.0)
        acc_s = acc_s + e

    @pl.when(i == 0)
    def _init():
        acc_ref[0 * c:1 * c, :] = acc_i
        acc_ref[1 * c:2 * c, :] = acc_s

    @pl.when(i != 0)
    def _accum():
        acc_ref[0 * c:1 * c, :] = acc_ref[0 * c:1 * c, :] + acc_i
        acc_ref[1 * c:2 * c, :] = acc_ref[1 * c:2 * c, :] + acc_s

    @pl.when(i == nblocks - 1)
    def _finish():
        isum = jnp.sum(acc_ref[0 * c:1 * c, :], axis=1, keepdims=True)
        ssum = jnp.sum(acc_ref[1 * c:2 * c, :], axis=1, keepdims=True)
        cnt_row = jnp.sum(cnt_ref[...], axis=0, keepdims=True)
        nsum = jnp.concatenate(
            [jnp.sum(cnt_row[:, cc * 16:(cc + 1) * 16], axis=1, keepdims=True)
             for cc in range(c)], axis=0)
        dice = (2.0 * isum) / (ssum + nsum + _EPS)
        w = (jax.lax.broadcasted_iota(jnp.int32, (c, 1), 0) != _IGNORE)
        out_ref[...] = jnp.sum(jnp.where(w, 1.0 - dice, 0.0), keepdims=True) / c


def kernel(output, target):
    n, c = output.shape
    nb = n // _BN
    xt = output.T
    t32 = target.astype(jnp.int32)
    t_flat = t32.reshape(n)
    t_lp = t32.reshape(n // 128, 128)

    mesh = plsc.VectorSubcoreMesh(core_axis_name="c", subcore_axis_name="s")
    sc_hist = functools.partial(
        pl.kernel,
        mesh=mesh,
        out_type=jax.ShapeDtypeStruct((_NW, _CPAD * 16), jnp.float32),
        scratch_types=[
            pltpu.VMEM((n // _NW,), jnp.int32),
            pltpu.VMEM((_CPAD * 16,), jnp.float32),
        ],
        compiler_params=pltpu.CompilerParams(needs_layout_passes=False),
    )(_sc_hist_body)
    counts = sc_hist(t_flat)

    loss = pl.pallas_call(
        functools.partial(_dice_body, nblocks=nb, c=c),
        grid=(nb,),
        in_specs=[
            pl.BlockSpec((c, _BN), lambda i: (0, i)),
            pl.BlockSpec((_BN // 128, 128), lambda i: (i, 0)),
            pl.BlockSpec((_NW, _CPAD * 16), lambda i: (0, 0)),
        ],
        out_specs=pl.BlockSpec((1, 1), lambda i: (0, 0)),
        out_shape=jax.ShapeDtypeStruct((1, 1), jnp.float32),
        scratch_shapes=[pltpu.VMEM((2 * c, 128), jnp.float32)],
        compiler_params=pltpu.CompilerParams(
            dimension_semantics=("arbitrary",),
        ),
    )(xt, t_lp, counts)
    return loss[0, 0]


# speedup vs baseline: 1.9663x; 1.9663x over previous
import functools

import jax
import jax.numpy as jnp
from jax.experimental import pallas as pl
from jax.experimental.pallas import tpu as pltpu

_IGNORE = 0
_EPS = 1e-6
_BN = 131072


def _dice_body(x_ref, t_ref, out_ref, acc_ref, *, nblocks, c):
    i = pl.program_id(0)
    cls = jax.lax.broadcasted_iota(jnp.int32, (c, 128), 0)

    acc_i = jnp.zeros((c, 128), dtype=jnp.float32)
    acc_s = jnp.zeros((c, 128), dtype=jnp.float32)
    acc_n = jnp.zeros((c, 128), dtype=jnp.float32)
    for j in range(_BN // 128):
        e = jnp.exp(x_ref[:, j * 128:(j + 1) * 128])
        m = t_ref[j:j + 1, :] == cls
        acc_i = acc_i + jnp.where(m, e, 0.0)
        acc_s = acc_s + e
        acc_n = acc_n + jnp.where(m, 1.0, 0.0)

    @pl.when(i == 0)
    def _init():
        acc_ref[0 * c:1 * c, :] = acc_i
        acc_ref[1 * c:2 * c, :] = acc_s
        acc_ref[2 * c:3 * c, :] = acc_n

    @pl.when(i != 0)
    def _accum():
        acc_ref[0 * c:1 * c, :] = acc_ref[0 * c:1 * c, :] + acc_i
        acc_ref[1 * c:2 * c, :] = acc_ref[1 * c:2 * c, :] + acc_s
        acc_ref[2 * c:3 * c, :] = acc_ref[2 * c:3 * c, :] + acc_n

    @pl.when(i == nblocks - 1)
    def _finish():
        isum = jnp.sum(acc_ref[0 * c:1 * c, :], axis=1, keepdims=True)
        ssum = jnp.sum(acc_ref[1 * c:2 * c, :], axis=1, keepdims=True)
        nsum = jnp.sum(acc_ref[2 * c:3 * c, :], axis=1, keepdims=True)
        dice = (2.0 * isum) / (ssum + nsum + _EPS)
        w = (jax.lax.broadcasted_iota(jnp.int32, (c, 1), 0) != _IGNORE)
        out_ref[...] = jnp.sum(jnp.where(w, 1.0 - dice, 0.0), keepdims=True) / c


def kernel(output, target):
    n, c = output.shape
    nb = n // _BN
    xt = output.T
    t_lp = target.astype(jnp.int32).reshape(n // 128, 128)
    loss = pl.pallas_call(
        functools.partial(_dice_body, nblocks=nb, c=c),
        grid=(nb,),
        in_specs=[
            pl.BlockSpec((c, _BN), lambda i: (0, i)),
            pl.BlockSpec((_BN // 128, 128), lambda i: (i, 0)),
        ],
        out_specs=pl.BlockSpec((1, 1), lambda i: (0, 0)),
        out_shape=jax.ShapeDtypeStruct((1, 1), jnp.float32),
        scratch_shapes=[pltpu.VMEM((3 * c, 128), jnp.float32)],
        compiler_params=pltpu.CompilerParams(
            dimension_semantics=("arbitrary",),
        ),
    )(xt, t_lp)
    return loss[0, 0]
